# baseline (speedup 1.0000x reference)
import functools
import math

import jax
import jax.numpy as jnp
from jax.experimental import pallas as pl
from jax.experimental.pallas import tpu as pltpu

D_MODEL_ = 256
POS_DIM_ = 32
_NC = 2


def _fourier_table(M, dtype):
    half = POS_DIM_ // 2
    freqs = jnp.exp(jnp.linspace(math.log(1.0), math.log(1000.0), half))
    idx = jnp.arange(M, dtype=dtype)
    pos = jnp.clip(idx / max(1, M - 1), 0.0, 1.0)
    ang = pos[:, None] * freqs
    out = jnp.concatenate([jnp.sin(ang), jnp.cos(ang)], axis=-1)
    if out.shape[-1] < POS_DIM_:
        out = jnp.concatenate(
            [out, jnp.zeros(out.shape[:-1] + (POS_DIM_ - out.shape[-1],), out.dtype)],
            axis=-1)
    return out.astype(dtype)


def _pool_kernel(bounds_ref, x_ref, w_ref, bias_ref, ff_ref, o_ref,
                 acc_ref, xbuf_ref, sem_ref, *, B, T, Tc):
    M = bounds_ref.shape[1]
    i = pl.program_id(0)
    n_steps = B * _NC
    c = jax.lax.rem(i, _NC)

    def _copy(step):
        b2 = jax.lax.div(step, _NC)
        c2 = jax.lax.rem(step, _NC)
        slot = jax.lax.rem(step, 2)
        return pltpu.make_async_copy(
            x_ref.at[b2, pl.ds(c2 * Tc, Tc), :],
            xbuf_ref.at[slot],
            sem_ref.at[slot],
        )

    @pl.when(i == 0)
    def _first():
        _copy(i).start()

    @pl.when(i + 1 < n_steps)
    def _prefetch():
        _copy(i + 1).start()

    _copy(i).wait()

    s = bounds_ref[0, :, 0]
    e = bounds_ref[0, :, 1]
    s = jnp.clip(s, 0, T - 1)
    e = jnp.minimum(e, T)
    e = jnp.maximum(s + 1, e)

    t16 = jax.lax.broadcasted_iota(jnp.int16, (M, Tc), 1)
    base = (c * Tc).astype(jnp.int16)
    s16 = s.astype(jnp.int16) - base
    e16 = e.astype(jnp.int16) - base
    mask = (t16 >= s16[:, None]) & (t16 < e16[:, None])
    maskf = jnp.where(mask, jnp.bfloat16(1.0), jnp.bfloat16(0.0))

    slot = jax.lax.rem(i, 2)
    part = jnp.dot(maskf, xbuf_ref[slot].astype(jnp.bfloat16),
                   preferred_element_type=jnp.float32)

    @pl.when(c == 0)
    def _init():
        acc_ref[...] = part

    @pl.when(c != 0)
    def _accum():
        acc_ref[...] += part

    @pl.when(c == _NC - 1)
    def _finish():
        inv = 1.0 / (e - s).astype(jnp.float32)
        mean = acc_ref[...] * inv[:, None]
        w_top = w_ref[:D_MODEL_, :]
        w_bot = w_ref[D_MODEL_:, :]
        out = jnp.dot(mean, w_top, preferred_element_type=jnp.float32)
        out += jnp.dot(ff_ref[...], w_bot, preferred_element_type=jnp.float32)
        out += bias_ref[...][None, :]
        o_ref[0] = out


def kernel(frame_emb, beat_bounds, W, b):
    B, T, D = frame_emb.shape
    M = beat_bounds.shape[1]
    Tc = T // _NC
    bounds = beat_bounds.astype(jnp.int32)
    ff = _fourier_table(M, frame_emb.dtype)

    return pl.pallas_call(
        functools.partial(_pool_kernel, B=B, T=T, Tc=Tc),
        grid=(B * _NC,),
        in_specs=[
            pl.BlockSpec((1, M, 2), lambda i: (i // _NC, 0, 0)),
            pl.BlockSpec(memory_space=pl.ANY),
            pl.BlockSpec((D + POS_DIM_, D), lambda i: (0, 0)),
            pl.BlockSpec((D,), lambda i: (0,)),
            pl.BlockSpec((M, POS_DIM_), lambda i: (0, 0)),
        ],
        out_specs=pl.BlockSpec((1, M, D), lambda i: (i // _NC, 0, 0)),
        out_shape=jax.ShapeDtypeStruct((B, M, D), frame_emb.dtype),
        scratch_shapes=[
            pltpu.VMEM((M, D), jnp.float32),
            pltpu.VMEM((2, Tc, D), jnp.float32),
            pltpu.SemaphoreType.DMA((2,)),
        ],
        compiler_params=pltpu.CompilerParams(
            dimension_semantics=("arbitrary",)),
    )(bounds, frame_emb, W, b, ff)


# speedup vs baseline: 1.3917x; 1.3917x over previous
import math

import jax
import jax.numpy as jnp
from jax.experimental import pallas as pl
from jax.experimental.pallas import tpu as pltpu

D_MODEL_ = 256
POS_DIM_ = 32


def _fourier_table(M, dtype):
    half = POS_DIM_ // 2
    freqs = jnp.exp(jnp.linspace(math.log(1.0), math.log(1000.0), half))
    idx = jnp.arange(M, dtype=dtype)
    pos = jnp.clip(idx / max(1, M - 1), 0.0, 1.0)
    ang = pos[:, None] * freqs
    out = jnp.concatenate([jnp.sin(ang), jnp.cos(ang)], axis=-1)
    if out.shape[-1] < POS_DIM_:
        out = jnp.concatenate(
            [out, jnp.zeros(out.shape[:-1] + (POS_DIM_ - out.shape[-1],), out.dtype)],
            axis=-1)
    return out.astype(dtype)


def _pool_kernel(bounds_ref, x_ref, w_ref, bias_ref, ff_ref, o_ref):
    T = x_ref.shape[1]
    M = bounds_ref.shape[1]
    i = pl.program_id(0)

    bnd = bounds_ref[pl.ds(i, 1), :, :]
    s = bnd[0, :, 0]
    e = bnd[0, :, 1]
    s = jnp.clip(s, 0, T - 1)
    e = jnp.minimum(e, T)
    e = jnp.maximum(s + 1, e)

    t16 = jax.lax.broadcasted_iota(jnp.int16, (M, T), 1)
    s16 = s.astype(jnp.int16)
    e16 = e.astype(jnp.int16)
    mask = (t16 >= s16[:, None]) & (t16 < e16[:, None])
    maskf = jnp.where(mask, jnp.bfloat16(1.0), jnp.bfloat16(0.0))

    sums = jnp.dot(maskf, x_ref[0].astype(jnp.bfloat16),
                   preferred_element_type=jnp.float32)
    inv = 1.0 / (e - s).astype(jnp.float32)
    mean = sums * inv[:, None]

    w_top = w_ref[:D_MODEL_, :]
    w_bot = w_ref[D_MODEL_:, :]
    out = jnp.dot(mean, w_top, preferred_element_type=jnp.float32)
    out += jnp.dot(ff_ref[...], w_bot, preferred_element_type=jnp.float32)
    out += bias_ref[...][None, :]
    o_ref[0] = out


def kernel(frame_emb, beat_bounds, W, b):
    B, T, D = frame_emb.shape
    M = beat_bounds.shape[1]
    bounds = beat_bounds.astype(jnp.int32)
    ff = _fourier_table(M, frame_emb.dtype)

    return pl.pallas_call(
        _pool_kernel,
        grid=(B,),
        in_specs=[
            pl.BlockSpec((B, M, 2), lambda i: (0, 0, 0)),
            pl.BlockSpec((1, T, D), lambda i: (i, 0, 0)),
            pl.BlockSpec((D + POS_DIM_, D), lambda i: (0, 0)),
            pl.BlockSpec((D,), lambda i: (0,)),
            pl.BlockSpec((M, POS_DIM_), lambda i: (0, 0)),
        ],
        out_specs=pl.BlockSpec((1, M, D), lambda i: (i, 0, 0)),
        out_shape=jax.ShapeDtypeStruct((B, M, D), frame_emb.dtype),
        compiler_params=pltpu.CompilerParams(
            dimension_semantics=("arbitrary",)),
    )(bounds, frame_emb, W, b, ff)
